# Initial kernel scaffold; baseline (speedup 1.0000x reference)
import jax
import jax.numpy as jnp
from jax.experimental import pallas as pl


def kernel(x, xe, ye, y, log_temp):
    raise NotImplementedError("write your pallas kernel here")


# speedup vs baseline: 6.7818x; 6.7818x over previous
import numpy as np
import jax
import jax.numpy as jnp
from jax.experimental import pallas as pl

PS = 10
STRIDE = 4
K = 7
T = 23
N = T * T
H = 98
CE = 32
CX = 3
FX = CX * PS * PS
NQ = 544
ND = 640
QBLK = 136

_PREC = jax.lax.Precision.HIGHEST


def _np_consts():
    S = np.zeros((T * PS, H), np.float32)
    for t in range(T):
        for a in range(PS):
            S[t * PS + a, STRIDE * t + a] = 1.0
    s1 = S.sum(axis=0)
    invw = (1.0 / (np.outer(s1, s1) + 1e-10)).astype(np.float32)
    return S, invw


_S_NP, _INVW_NP = _np_consts()


def _dot(a, b, dims):
    return jax.lax.dot_general(a, b, (dims, ((), ())), precision=_PREC,
                               preferred_element_type=jnp.float32)


def _unfold_mm(s, img):
    t1 = _dot(s, img, ((1,), (0,)))
    return _dot(t1, s, ((1,), (1,)))


def _kernel_a(xe_ref, ye_ref, s_ref, ux_ref, uy_ref):
    s = s_ref[...]
    ux_ref[0] = _unfold_mm(s, xe_ref[0])
    uy_ref[0] = _unfold_mm(s, ye_ref[0])


def _kernel_b(x_ref, lt_ref, s_ref, uxp_ref, ult_ref):
    s = s_ref[...]
    for c in range(CX):
        uxp_ref[c] = _unfold_mm(s, x_ref[c])
    ult_ref[0] = _unfold_mm(s, lt_ref[0])


def _kernel_n(xe_ref, dn_ref):
    xe_p = xe_ref[...]
    dn_ref[...] = jnp.sum(xe_p ** 2, axis=1, keepdims=True)


def _kernel_c(ye_ref, lt_ref, xe_ref, dn_ref, xp_ref, out_ref):
    ye_p = ye_ref[...]
    xe_p = xe_ref[...]
    g = jax.lax.dot_general(ye_p, xe_p, (((1,), (1,)), ((), ())),
                            preferred_element_type=jnp.float32)
    d2 = (jnp.sum(ye_p ** 2, axis=1, keepdims=True)
          + dn_ref[...]
          - 2.0 * g)
    lt_p = jnp.mean(lt_ref[...], axis=1, keepdims=True)
    logits = (-d2) / jnp.exp(lt_p)
    mask = jax.lax.broadcasted_iota(jnp.int32, (1, ND), 1) >= N
    lg = jnp.where(mask, -1e30, logits)
    xp = xp_ref[...]
    for k in range(K):
        m = jnp.max(lg, axis=-1, keepdims=True)
        un = jnp.exp(lg - m)
        w = un / jnp.sum(un, axis=-1, keepdims=True)
        out_ref[k] = jax.lax.dot_general(w, xp, (((1,), (0,)), ((), ())),
                                         preferred_element_type=jnp.float32)
        if k < K - 1:
            lg = lg + jnp.log(jnp.clip(1.0 - w, 1e-10, None))


def _kernel_d(z_ref, s_ref, invw_ref, yp_ref, o_ref):
    s = s_ref[...]
    invw = invw_ref[...]
    for kc in range(K * CX):
        t1 = _dot(s, z_ref[kc], ((0,), (0,)))
        v = _dot(t1, s, ((1,), (0,)))
        o_ref[kc] = v * invw - yp_ref[kc % CX]


def kernel(x, xe, ye, y, log_temp):
    f32 = jnp.float32
    pad = lambda v: jnp.pad(v, ((0, 0), (0, 0), (1, 1), (1, 1)))[0]
    xp_img = pad(x)
    xe_img = pad(xe)
    ye_img = pad(ye)
    yp_img = pad(y)
    lt_img = pad(log_temp)

    s_c = jnp.asarray(_S_NP)
    invw_c = jnp.asarray(_INVW_NP)

    ux, uy = pl.pallas_call(
        _kernel_a,
        grid=(CE,),
        in_specs=[
            pl.BlockSpec((1, H, H), lambda i: (i, 0, 0)),
            pl.BlockSpec((1, H, H), lambda i: (i, 0, 0)),
            pl.BlockSpec((T * PS, H), lambda i: (0, 0)),
        ],
        out_specs=[
            pl.BlockSpec((1, T * PS, T * PS), lambda i: (i, 0, 0)),
            pl.BlockSpec((1, T * PS, T * PS), lambda i: (i, 0, 0)),
        ],
        out_shape=[
            jax.ShapeDtypeStruct((CE, T * PS, T * PS), f32),
            jax.ShapeDtypeStruct((CE, T * PS, T * PS), f32),
        ],
    )(xe_img, ye_img, s_c)

    uxp, ult = pl.pallas_call(
        _kernel_b,
        out_shape=[
            jax.ShapeDtypeStruct((CX, T * PS, T * PS), f32),
            jax.ShapeDtypeStruct((1, T * PS, T * PS), f32),
        ],
    )(xp_img, lt_img, s_c)

    def to_patch_rows(u, c):
        u5 = u.reshape(c, T, PS, T, PS).transpose(1, 3, 0, 2, 4)
        return u5.reshape(N, c * PS * PS)

    xe_p = jnp.pad(to_patch_rows(ux, CE), ((0, ND - N), (0, 0)))
    ye_p = jnp.pad(to_patch_rows(uy, CE), ((0, NQ - N), (0, 0)))
    x_p = jnp.pad(to_patch_rows(uxp, CX), ((0, ND - N), (0, 0)))
    lt_u = jnp.pad(to_patch_rows(ult, 1), ((0, NQ - N), (0, 0)))

    dn_col = pl.pallas_call(
        _kernel_n,
        out_shape=jax.ShapeDtypeStruct((ND, 1), f32),
    )(xe_p)
    dn_row = dn_col.reshape(1, ND)

    zp = pl.pallas_call(
        _kernel_c,
        grid=(NQ // QBLK,),
        in_specs=[
            pl.BlockSpec((QBLK, CE * PS * PS), lambda i: (i, 0)),
            pl.BlockSpec((QBLK, PS * PS), lambda i: (i, 0)),
            pl.BlockSpec((ND, CE * PS * PS), lambda i: (0, 0)),
            pl.BlockSpec((1, ND), lambda i: (0, 0)),
            pl.BlockSpec((ND, FX), lambda i: (0, 0)),
        ],
        out_specs=pl.BlockSpec((K, QBLK, FX), lambda i: (0, i, 0)),
        out_shape=jax.ShapeDtypeStruct((K, NQ, FX), f32),
    )(ye_p, lt_u, xe_p, dn_row, x_p)

    z6 = zp[:, :N, :].reshape(K, T, T, CX, PS, PS).transpose(0, 3, 1, 4, 2, 5)
    zf = z6.reshape(K * CX, T * PS, T * PS)

    out = pl.pallas_call(
        _kernel_d,
        out_shape=jax.ShapeDtypeStruct((K * CX, H, H), f32),
    )(zf, s_c, invw_c, yp_img)

    z = jnp.concatenate([yp_img[None], out[None]], axis=1)
    return z[..., 1:-1, 1:-1]


# speedup vs baseline: 9.2811x; 1.3685x over previous
import numpy as np
import jax
import jax.numpy as jnp
from jax.experimental import pallas as pl

PS = 10
STRIDE = 4
K = 7
T = 23
N = T * T
H = 98
CE = 32
CX = 3
FX = CX * PS * PS
NQ = 544
ND = 640
QBLK = 136

_PREC = jax.lax.Precision.HIGHEST


def _np_consts():
    S = np.zeros((T * PS, H), np.float32)
    for t in range(T):
        for a in range(PS):
            S[t * PS + a, STRIDE * t + a] = 1.0
    s1 = S.sum(axis=0)
    invw = (1.0 / (np.outer(s1, s1) + 1e-10)).astype(np.float32)
    return S, invw


_S_NP, _INVW_NP = _np_consts()


def _dot(a, b, dims):
    return jax.lax.dot_general(a, b, (dims, ((), ())), precision=_PREC,
                               preferred_element_type=jnp.float32)


def _unfold_mm(s, img):
    t1 = _dot(s, img, ((1,), (0,)))
    return _dot(t1, s, ((1,), (1,)))


def _patch_rows(u, nrows):
    r = u.reshape(T, PS, T, PS).transpose(0, 2, 1, 3).reshape(N, PS * PS)
    return jnp.concatenate(
        [r, jnp.zeros((nrows - N, PS * PS), jnp.float32)], axis=0)


def _kernel_a(xe_ref, ye_ref, s_ref, xep_ref, yep_ref):
    s = s_ref[...]
    p = PS * PS
    for c in range(CE):
        xep_ref[:, c * p:(c + 1) * p] = _patch_rows(_unfold_mm(s, xe_ref[c]), ND)
        yep_ref[:, c * p:(c + 1) * p] = _patch_rows(_unfold_mm(s, ye_ref[c]), NQ)


def _kernel_b(x_ref, lt_ref, s_ref, uxp_ref, ult_ref):
    s = s_ref[...]
    for c in range(CX):
        uxp_ref[c] = _unfold_mm(s, x_ref[c])
    ult_ref[0] = _unfold_mm(s, lt_ref[0])


def _kernel_n(xe_ref, dn_ref):
    xe_p = xe_ref[...]
    dn_ref[...] = jnp.sum(xe_p ** 2, axis=1, keepdims=True)


def _kernel_c(ye_ref, lt_ref, xe_ref, dn_ref, xp_ref, out_ref):
    ye_p = ye_ref[...]
    xe_p = xe_ref[...]
    g = jax.lax.dot_general(ye_p, xe_p, (((1,), (1,)), ((), ())),
                            preferred_element_type=jnp.float32)
    d2 = (jnp.sum(ye_p ** 2, axis=1, keepdims=True)
          + dn_ref[...]
          - 2.0 * g)
    lt_p = jnp.mean(lt_ref[...], axis=1, keepdims=True)
    logits = (-d2) / jnp.exp(lt_p)
    mask = jax.lax.broadcasted_iota(jnp.int32, (1, ND), 1) >= N
    lg = jnp.where(mask, -1e30, logits)
    xp = xp_ref[...]
    for k in range(K):
        m = jnp.max(lg, axis=-1, keepdims=True)
        un = jnp.exp(lg - m)
        w = un / jnp.sum(un, axis=-1, keepdims=True)
        out_ref[k] = jax.lax.dot_general(w, xp, (((1,), (0,)), ((), ())),
                                         preferred_element_type=jnp.float32)
        if k < K - 1:
            lg = lg + jnp.log(jnp.clip(1.0 - w, 1e-10, None))


def _kernel_d(z_ref, s_ref, invw_ref, yp_ref, o_ref):
    s = s_ref[...]
    invw = invw_ref[...]
    for kc in range(K * CX):
        t1 = _dot(s, z_ref[kc], ((0,), (0,)))
        v = _dot(t1, s, ((1,), (0,)))
        o_ref[kc] = v * invw - yp_ref[kc % CX]


def kernel(x, xe, ye, y, log_temp):
    f32 = jnp.float32
    pad = lambda v: jnp.pad(v, ((0, 0), (0, 0), (1, 1), (1, 1)))[0]
    xp_img = pad(x)
    xe_img = pad(xe)
    ye_img = pad(ye)
    yp_img = pad(y)
    lt_img = pad(log_temp)

    s_c = jnp.asarray(_S_NP)
    invw_c = jnp.asarray(_INVW_NP)

    xe_p, ye_p = pl.pallas_call(
        _kernel_a,
        out_shape=[
            jax.ShapeDtypeStruct((ND, CE * PS * PS), f32),
            jax.ShapeDtypeStruct((NQ, CE * PS * PS), f32),
        ],
    )(xe_img, ye_img, s_c)

    uxp, ult = pl.pallas_call(
        _kernel_b,
        out_shape=[
            jax.ShapeDtypeStruct((CX, T * PS, T * PS), f32),
            jax.ShapeDtypeStruct((1, T * PS, T * PS), f32),
        ],
    )(xp_img, lt_img, s_c)

    def to_patch_rows(u, c):
        u5 = u.reshape(c, T, PS, T, PS).transpose(1, 3, 0, 2, 4)
        return u5.reshape(N, c * PS * PS)

    x_p = jnp.pad(to_patch_rows(uxp, CX), ((0, ND - N), (0, 0)))
    lt_u = jnp.pad(to_patch_rows(ult, 1), ((0, NQ - N), (0, 0)))

    dn_col = pl.pallas_call(
        _kernel_n,
        out_shape=jax.ShapeDtypeStruct((ND, 1), f32),
    )(xe_p)
    dn_row = dn_col.reshape(1, ND)

    zp = pl.pallas_call(
        _kernel_c,
        grid=(NQ // QBLK,),
        in_specs=[
            pl.BlockSpec((QBLK, CE * PS * PS), lambda i: (i, 0)),
            pl.BlockSpec((QBLK, PS * PS), lambda i: (i, 0)),
            pl.BlockSpec((ND, CE * PS * PS), lambda i: (0, 0)),
            pl.BlockSpec((1, ND), lambda i: (0, 0)),
            pl.BlockSpec((ND, FX), lambda i: (0, 0)),
        ],
        out_specs=pl.BlockSpec((K, QBLK, FX), lambda i: (0, i, 0)),
        out_shape=jax.ShapeDtypeStruct((K, NQ, FX), f32),
    )(ye_p, lt_u, xe_p, dn_row, x_p)

    z6 = zp[:, :N, :].reshape(K, T, T, CX, PS, PS).transpose(0, 3, 1, 4, 2, 5)
    zf = z6.reshape(K * CX, T * PS, T * PS)

    out = pl.pallas_call(
        _kernel_d,
        out_shape=jax.ShapeDtypeStruct((K * CX, H, H), f32),
    )(zf, s_c, invw_c, yp_img)

    z = jnp.concatenate([yp_img[None], out[None]], axis=1)
    return z[..., 1:-1, 1:-1]


# speedup vs baseline: 12.6168x; 1.3594x over previous
import numpy as np
import jax
import jax.numpy as jnp
from jax.experimental import pallas as pl

PS = 10
STRIDE = 4
K = 7
T = 23
N = T * T
H = 98
CE = 32
CX = 3
FX = CX * PS * PS
NQ = 544
ND = 640
QBLK = 136

_PREC = jax.lax.Precision.HIGHEST


def _np_consts():
    S = np.zeros((T * PS, H), np.float32)
    for t in range(T):
        for a in range(PS):
            S[t * PS + a, STRIDE * t + a] = 1.0
    s1 = S.sum(axis=0)
    invw = (1.0 / (np.outer(s1, s1) + 1e-10)).astype(np.float32)
    return S, invw


_S_NP, _INVW_NP = _np_consts()


def _dot(a, b, dims):
    return jax.lax.dot_general(a, b, (dims, ((), ())), precision=_PREC,
                               preferred_element_type=jnp.float32)


def _unfold_mm(s, img):
    t1 = _dot(s, img, ((1,), (0,)))
    return _dot(t1, s, ((1,), (1,)))


def _patch_rows(u, nrows):
    r = u.reshape(T, PS, T, PS).transpose(0, 2, 1, 3).reshape(N, PS * PS)
    return jnp.concatenate(
        [r, jnp.zeros((nrows - N, PS * PS), jnp.float32)], axis=0)


def _kernel_a(xe_ref, ye_ref, s_ref, xep_ref, yep_ref):
    s = s_ref[...]
    p = PS * PS
    for c in range(CE):
        xep_ref[:, c * p:(c + 1) * p] = _patch_rows(_unfold_mm(s, xe_ref[c]), ND)
        yep_ref[:, c * p:(c + 1) * p] = _patch_rows(_unfold_mm(s, ye_ref[c]), NQ)


def _kernel_b(x_ref, lt_ref, s_ref, xp_ref, ltu_ref):
    s = s_ref[...]
    p = PS * PS
    for c in range(CX):
        xp_ref[:, c * p:(c + 1) * p] = _patch_rows(_unfold_mm(s, x_ref[c]), ND)
    ltu_ref[...] = _patch_rows(_unfold_mm(s, lt_ref[0]), NQ)


def _kernel_c(ye_ref, lt_ref, xe_ref, xp_ref, out_ref):
    ye_p = ye_ref[...]
    xe_p = xe_ref[...]
    g = jax.lax.dot_general(ye_p, xe_p, (((1,), (1,)), ((), ())),
                            preferred_element_type=jnp.float32)
    d2 = (jnp.sum(ye_p ** 2, axis=1, keepdims=True)
          + jnp.sum(xe_p ** 2, axis=1)[None, :]
          - 2.0 * g)
    lt_p = jnp.mean(lt_ref[...], axis=1, keepdims=True)
    logits = (-d2) / jnp.exp(lt_p)
    mask = jax.lax.broadcasted_iota(jnp.int32, (1, ND), 1) >= N
    lg = jnp.where(mask, -1e30, logits)
    xp = xp_ref[...]
    for k in range(K):
        m = jnp.max(lg, axis=-1, keepdims=True)
        un = jnp.exp(lg - m)
        w = un / jnp.sum(un, axis=-1, keepdims=True)
        out_ref[k] = jax.lax.dot_general(w, xp, (((1,), (0,)), ((), ())),
                                         preferred_element_type=jnp.float32)
        if k < K - 1:
            lg = lg + jnp.log(jnp.clip(1.0 - w, 1e-10, None))


def _kernel_d(z_ref, s_ref, invw_ref, yp_ref, o_ref):
    s = s_ref[...]
    invw = invw_ref[...]
    for k in range(K):
        z5 = z_ref[k][0:N, :].reshape(T, T, CX, PS, PS)
        for c in range(CX):
            m = z5[:, :, c, :, :].transpose(0, 2, 1, 3).reshape(T * PS, T * PS)
            t1 = _dot(s, m, ((0,), (0,)))
            v = _dot(t1, s, ((1,), (0,)))
            o_ref[k * CX + c] = v * invw - yp_ref[c]


def kernel(x, xe, ye, y, log_temp):
    f32 = jnp.float32
    pad = lambda v: jnp.pad(v, ((0, 0), (0, 0), (1, 1), (1, 1)))[0]
    xp_img = pad(x)
    xe_img = pad(xe)
    ye_img = pad(ye)
    yp_img = pad(y)
    lt_img = pad(log_temp)

    s_c = jnp.asarray(_S_NP)
    invw_c = jnp.asarray(_INVW_NP)

    xe_p, ye_p = pl.pallas_call(
        _kernel_a,
        out_shape=[
            jax.ShapeDtypeStruct((ND, CE * PS * PS), f32),
            jax.ShapeDtypeStruct((NQ, CE * PS * PS), f32),
        ],
    )(xe_img, ye_img, s_c)

    x_p, lt_u = pl.pallas_call(
        _kernel_b,
        out_shape=[
            jax.ShapeDtypeStruct((ND, FX), f32),
            jax.ShapeDtypeStruct((NQ, PS * PS), f32),
        ],
    )(xp_img, lt_img, s_c)

    zp = pl.pallas_call(
        _kernel_c,
        grid=(NQ // QBLK,),
        in_specs=[
            pl.BlockSpec((QBLK, CE * PS * PS), lambda i: (i, 0)),
            pl.BlockSpec((QBLK, PS * PS), lambda i: (i, 0)),
            pl.BlockSpec((ND, CE * PS * PS), lambda i: (0, 0)),
            pl.BlockSpec((ND, FX), lambda i: (0, 0)),
        ],
        out_specs=pl.BlockSpec((K, QBLK, FX), lambda i: (0, i, 0)),
        out_shape=jax.ShapeDtypeStruct((K, NQ, FX), f32),
    )(ye_p, lt_u, xe_p, x_p)

    out = pl.pallas_call(
        _kernel_d,
        out_shape=jax.ShapeDtypeStruct((K * CX, H, H), f32),
    )(zp, s_c, invw_c, yp_img)

    z = jnp.concatenate([yp_img[None], out[None]], axis=1)
    return z[..., 1:-1, 1:-1]


# speedup vs baseline: 13.4127x; 1.0631x over previous
import numpy as np
import jax
import jax.numpy as jnp
from jax.experimental import pallas as pl
from jax.experimental.pallas import tpu as pltpu

PS = 10
STRIDE = 4
K = 7
T = 23
N = T * T
H = 98
CE = 32
CX = 3
FE = CE * PS * PS
FX = CX * PS * PS

_PREC = jax.lax.Precision.HIGHEST


def _np_consts():
    S = np.zeros((T * PS, H), np.float32)
    for t in range(T):
        for a in range(PS):
            S[t * PS + a, STRIDE * t + a] = 1.0
    s1 = S.sum(axis=0)
    invw = (1.0 / (np.outer(s1, s1) + 1e-10)).astype(np.float32)
    return S, invw


_S_NP, _INVW_NP = _np_consts()


def _dot(a, b, dims):
    return jax.lax.dot_general(a, b, (dims, ((), ())), precision=_PREC,
                               preferred_element_type=jnp.float32)


def _unfold_mm(s, img):
    t1 = _dot(s, img, ((1,), (0,)))
    return _dot(t1, s, ((1,), (1,)))


def _patch_rows(u):
    return u.reshape(T, PS, T, PS).transpose(0, 2, 1, 3).reshape(N, PS * PS)


def _mega(xe_ref, ye_ref, x_ref, lt_ref, yp_ref, s_ref, invw_ref,
          o_ref, xep_s, yep_s, xp_s):
    s = s_ref[...]
    p = PS * PS
    for c in range(CE):
        xep_s[:, c * p:(c + 1) * p] = _patch_rows(_unfold_mm(s, xe_ref[c]))
        yep_s[:, c * p:(c + 1) * p] = _patch_rows(_unfold_mm(s, ye_ref[c]))
    for c in range(CX):
        xp_s[:, c * p:(c + 1) * p] = _patch_rows(_unfold_mm(s, x_ref[c]))
    lt_u = _patch_rows(_unfold_mm(s, lt_ref[0]))

    ye_p = yep_s[...]
    xe_p = xep_s[...]
    g = jax.lax.dot_general(ye_p, xe_p, (((1,), (1,)), ((), ())),
                            preferred_element_type=jnp.float32)
    d2 = (jnp.sum(ye_p ** 2, axis=1, keepdims=True)
          + jnp.sum(xe_p ** 2, axis=1)[None, :]
          - 2.0 * g)
    lt_p = jnp.mean(lt_u, axis=1, keepdims=True)
    lg = (-d2) / jnp.exp(lt_p)

    xp = xp_s[...]
    invw = invw_ref[...]
    for k in range(K):
        m = jnp.max(lg, axis=-1, keepdims=True)
        un = jnp.exp(lg - m)
        w = un / jnp.sum(un, axis=-1, keepdims=True)
        zk = jax.lax.dot_general(w, xp, (((1,), (0,)), ((), ())),
                                 preferred_element_type=jnp.float32)
        z5 = zk.reshape(T, T, CX, PS, PS)
        for c in range(CX):
            zm = z5[:, :, c, :, :].transpose(0, 2, 1, 3).reshape(T * PS, T * PS)
            t1 = _dot(s, zm, ((0,), (0,)))
            v = _dot(t1, s, ((1,), (0,)))
            o_ref[k * CX + c] = v * invw - yp_ref[c]
        if k < K - 1:
            lg = lg + jnp.log(jnp.clip(1.0 - w, 1e-10, None))


def kernel(x, xe, ye, y, log_temp):
    f32 = jnp.float32
    pad = lambda v: jnp.pad(v, ((0, 0), (0, 0), (1, 1), (1, 1)))[0]
    xp_img = pad(x)
    xe_img = pad(xe)
    ye_img = pad(ye)
    yp_img = pad(y)
    lt_img = pad(log_temp)

    out = pl.pallas_call(
        _mega,
        out_shape=jax.ShapeDtypeStruct((K * CX, H, H), f32),
        scratch_shapes=[
            pltpu.VMEM((N, FE), f32),
            pltpu.VMEM((N, FE), f32),
            pltpu.VMEM((N, FX), f32),
        ],
    )(xe_img, ye_img, xp_img, lt_img, yp_img,
      jnp.asarray(_S_NP), jnp.asarray(_INVW_NP))

    z = jnp.concatenate([yp_img[None], out[None]], axis=1)
    return z[..., 1:-1, 1:-1]
